# baseline (speedup 1.0000x reference)
import jax
import jax.numpy as jnp
from jax.experimental import pallas as pl

_EPS = 1e-5


def _head_kernel(x_ref, w_ref, t_ref, o_ref):
    y = jnp.dot(w_ref[...], x_ref[0], preferred_element_type=jnp.float32)
    o_ref[0] = jnp.maximum(y + t_ref[...], 0.0).astype(jnp.bfloat16)


def kernel(xyz, params):
    W, bb, g, be = params["head"][0]
    s = g / jnp.sqrt(1.0 + _EPS)
    wt = W * s[:, None]
    t = (bb * s + be)[:, None]
    B, C, N = xyz.shape
    O = W.shape[0]
    y = pl.pallas_call(
        _head_kernel,
        grid=(B,),
        in_specs=[
            pl.BlockSpec((1, C, N), lambda b: (b, 0, 0)),
            pl.BlockSpec((O, C), lambda b: (0, 0)),
            pl.BlockSpec((O, 1), lambda b: (0, 0)),
        ],
        out_specs=pl.BlockSpec((1, O, N), lambda b: (b, 0, 0)),
        out_shape=jax.ShapeDtypeStruct((B, O, N), jnp.bfloat16),
    )(xyz, wt, t)
    return jnp.transpose(y, (0, 2, 1)).astype(xyz.dtype)


# speedup vs baseline: 1.2614x; 1.2614x over previous
import jax
import jax.numpy as jnp
from jax.experimental import pallas as pl

_EPS = 1e-5


def _head_kernel(x_ref, w_ref, t_ref, o_ref):
    y = jnp.dot(w_ref[...], x_ref[0], preferred_element_type=jnp.float32)
    o_ref[0] = jnp.maximum(y + t_ref[...], 0.0)


def kernel(xyz, params):
    W, bb, g, be = params["head"][0]
    s = g / jnp.sqrt(1.0 + _EPS)
    wt = W * s[:, None]
    t = (bb * s + be)[:, None]
    B, C, N = xyz.shape
    O = W.shape[0]
    y = pl.pallas_call(
        _head_kernel,
        grid=(B,),
        in_specs=[
            pl.BlockSpec((1, C, N), lambda b: (b, 0, 0)),
            pl.BlockSpec((O, C), lambda b: (0, 0)),
            pl.BlockSpec((O, 1), lambda b: (0, 0)),
        ],
        out_specs=pl.BlockSpec((1, O, N), lambda b: (b, 0, 0)),
        out_shape=jax.ShapeDtypeStruct((B, O, N), xyz.dtype),
    )(xyz, wt, t)
    return jnp.transpose(y, (0, 2, 1))
